# baseline (speedup 1.0000x reference)
import functools

import numpy as np
import jax
import jax.numpy as jnp
from jax import lax
from jax.experimental import pallas as pl
from jax.experimental.pallas import tpu as pltpu
from jax.experimental.pallas import tpu_sc as plsc

_B = 16384
_EMB = 64
_NC, _NS = 2, 16
_NW = _NC * _NS
_BPW = _B // _NW
_CH = 128
_NCHUNK = _BPW // _CH

_TILE = 2048


def _sc_gather_body(u_tab, c_tab, uids, cids, u_out, c_out,
                    idx_u, idx_c, rows, sem):
    wid = lax.axis_index("s") * _NC + lax.axis_index("c")
    base = wid * _BPW
    pltpu.sync_copy(uids.at[wid], idx_u)
    pltpu.sync_copy(cids.at[wid], idx_c)
    for tab, idx, out in ((u_tab, idx_u, u_out), (c_tab, idx_c, c_out)):
        copies = [pltpu.async_copy(
            tab.at[idx.at[j]], rows.at[pl.ds(j * _CH, _CH)], sem)
            for j in range(_NCHUNK)]
        for cp in copies:
            cp.wait()
        pltpu.sync_copy(rows, out.at[pl.ds(base, _BPW)])


@functools.cache
def _sc_gather():
    return pl.kernel(
        _sc_gather_body,
        out_type=(jax.ShapeDtypeStruct((_B, 2 * _EMB), jnp.float32),
                  jax.ShapeDtypeStruct((_B, 2 * _EMB), jnp.float32)),
        mesh=plsc.VectorSubcoreMesh(core_axis_name="c", subcore_axis_name="s"),
        scratch_types=[
            pltpu.VMEM((_NCHUNK, _CH), jnp.int32),
            pltpu.VMEM((_NCHUNK, _CH), jnp.int32),
            pltpu.VMEM((_BPW, 2 * _EMB), jnp.float32),
            pltpu.SemaphoreType.DMA,
        ],
    )


def _mlp_body(u_ref, c_ref, up_ref, cp_ref, w0u, w0c, w1, w2, w3,
              b0, b1, b2, b3, g0, g1, g2, be0, be1, be2, out_ref):
    s = np.float32(1.0 / np.sqrt(1.0 + 1e-5))

    def dot_t(x, w):
        return lax.dot_general(x, w, (((1,), (1,)), ((), ())),
                               preferred_element_type=jnp.float32)

    up = up_ref[...]
    cp = cp_ref[...]
    u2 = u_ref[...]
    c2 = c_ref[...]
    u = u2[:, :_EMB] + up * (u2[:, _EMB:] - u2[:, :_EMB])
    c = c2[:, :_EMB] + cp * (c2[:, _EMB:] - c2[:, :_EMB])
    h = dot_t(u, w0u[...]) + dot_t(c, w0c[...])
    h = jnp.maximum(h + b0[...], 0.0) * (g0[...] * s) + be0[...]
    h = dot_t(h, w1[...])
    h = jnp.maximum(h + b1[...], 0.0) * (g1[...] * s) + be1[...]
    h = dot_t(h, w2[...])
    h = jnp.maximum(h + b2[...], 0.0) * (g2[...] * s) + be2[...]
    h = dot_t(h, w3[...]) + b3[...]
    out_ref[...] = jax.nn.sigmoid(h)


def _mlp_call(u_emb, c_emb, upar, cpar, wb):
    full = lambda shape: pl.BlockSpec(shape, lambda i: (0, 0))
    return pl.pallas_call(
        _mlp_body,
        grid=(_B // _TILE,),
        in_specs=[
            pl.BlockSpec((_TILE, 2 * _EMB), lambda i: (i, 0)),
            pl.BlockSpec((_TILE, 2 * _EMB), lambda i: (i, 0)),
            pl.BlockSpec((_TILE, 1), lambda i: (i, 0)),
            pl.BlockSpec((_TILE, 1), lambda i: (i, 0)),
        ] + [full(w.shape) for w in wb],
        out_specs=pl.BlockSpec((_TILE, 8), lambda i: (i, 0)),
        out_shape=jax.ShapeDtypeStruct((_B, 8), jnp.float32),
    )(u_emb, c_emb, upar, cpar, *wb)


def kernel(user_ids, course_ids, params):
    user_ids = jnp.asarray(user_ids, jnp.int32)
    course_ids = jnp.asarray(course_ids, jnp.int32)
    u_tab = params['user_table'].reshape(-1, 2 * _EMB)
    c_tab = params['course_table'].reshape(-1, 2 * _EMB)
    uids = (user_ids >> 1).reshape(_NW, _NCHUNK, _CH)
    cids = (course_ids >> 1).reshape(_NW, _NCHUNK, _CH)
    u_emb, c_emb = _sc_gather()(u_tab, c_tab, uids, cids)
    upar = (user_ids & 1).astype(jnp.float32).reshape(_B, 1)
    cpar = (course_ids & 1).astype(jnp.float32).reshape(_B, 1)
    p = params
    row = lambda v: v.reshape(1, -1)
    w3p = jnp.pad(p['W3'], ((0, 7), (0, 0)))
    b3p = jnp.pad(p['b3'], (0, 7)).reshape(1, 8)
    wb = (p['W0'][:, :_EMB], p['W0'][:, _EMB:], p['W1'], p['W2'], w3p,
          row(p['b0']), row(p['b1']), row(p['b2']), b3p,
          row(p['gamma0']), row(p['gamma1']), row(p['gamma2']),
          row(p['beta0']), row(p['beta1']), row(p['beta2']))
    out = _mlp_call(u_emb, c_emb, upar, cpar, wb)
    return out[:, 0]


# speedup vs baseline: 1.0061x; 1.0061x over previous
import functools

import numpy as np
import jax
import jax.numpy as jnp
from jax import lax
from jax.experimental import pallas as pl
from jax.experimental.pallas import tpu as pltpu
from jax.experimental.pallas import tpu_sc as plsc

_B = 16384
_EMB = 64
_NC, _NS = 2, 16
_NW = _NC * _NS
_BPW = _B // _NW
_CH = 128
_NCHUNK = _BPW // _CH

_TILE = 2048


def _sc_gather_body(u_tab, c_tab, uids, cids, u_out, c_out,
                    idx_u, idx_c, rows_u, rows_c, sem):
    wid = lax.axis_index("s") * _NC + lax.axis_index("c")
    base = wid * _BPW
    pltpu.sync_copy(uids.at[wid], idx_u)
    pltpu.sync_copy(cids.at[wid], idx_c)
    copies = []
    for j in range(_NCHUNK):
        copies.append(pltpu.async_copy(
            u_tab.at[idx_u.at[j]], rows_u.at[pl.ds(j * _CH, _CH)], sem))
        copies.append(pltpu.async_copy(
            c_tab.at[idx_c.at[j]], rows_c.at[pl.ds(j * _CH, _CH)], sem))
    for cp in copies:
        cp.wait()
    pltpu.sync_copy(rows_u, u_out.at[pl.ds(base, _BPW)])
    pltpu.sync_copy(rows_c, c_out.at[pl.ds(base, _BPW)])


@functools.cache
def _sc_gather():
    return pl.kernel(
        _sc_gather_body,
        out_type=(jax.ShapeDtypeStruct((_B, _EMB), jnp.float32),
                  jax.ShapeDtypeStruct((_B, _EMB), jnp.float32)),
        mesh=plsc.VectorSubcoreMesh(core_axis_name="c", subcore_axis_name="s"),
        compiler_params=pltpu.CompilerParams(use_tc_tiling_on_sc=False),
        scratch_types=[
            pltpu.VMEM((_NCHUNK, _CH), jnp.int32),
            pltpu.VMEM((_NCHUNK, _CH), jnp.int32),
            pltpu.VMEM((_BPW, _EMB), jnp.float32),
            pltpu.VMEM((_BPW, _EMB), jnp.float32),
            pltpu.SemaphoreType.DMA,
        ],
    )


def _mlp_body(u_ref, c_ref, w0u, w0c, w1, w2, w3,
              b0, b1, b2, b3, g0, g1, g2, be0, be1, be2, out_ref):
    s = np.float32(1.0 / np.sqrt(1.0 + 1e-5))

    def dot_t(x, w):
        return lax.dot_general(x, w, (((1,), (1,)), ((), ())),
                               preferred_element_type=jnp.float32)

    h = dot_t(u_ref[...], w0u[...]) + dot_t(c_ref[...], w0c[...])
    h = jnp.maximum(h + b0[...], 0.0) * (g0[...] * s) + be0[...]
    h = dot_t(h, w1[...])
    h = jnp.maximum(h + b1[...], 0.0) * (g1[...] * s) + be1[...]
    h = dot_t(h, w2[...])
    h = jnp.maximum(h + b2[...], 0.0) * (g2[...] * s) + be2[...]
    h = dot_t(h, w3[...]) + b3[...]
    out_ref[...] = jax.nn.sigmoid(h)


def _mlp_call(u_emb, c_emb, wb):
    full = lambda shape: pl.BlockSpec(shape, lambda i: (0, 0))
    return pl.pallas_call(
        _mlp_body,
        grid=(_B // _TILE,),
        in_specs=[
            pl.BlockSpec((_TILE, _EMB), lambda i: (i, 0)),
            pl.BlockSpec((_TILE, _EMB), lambda i: (i, 0)),
        ] + [full(w.shape) for w in wb],
        out_specs=pl.BlockSpec((_TILE, 8), lambda i: (i, 0)),
        out_shape=jax.ShapeDtypeStruct((_B, 8), jnp.float32),
    )(u_emb, c_emb, *wb)


def kernel(user_ids, course_ids, params):
    uids = jnp.asarray(user_ids, jnp.int32).reshape(_NW, _NCHUNK, _CH)
    cids = jnp.asarray(course_ids, jnp.int32).reshape(_NW, _NCHUNK, _CH)
    u_emb, c_emb = _sc_gather()(params['user_table'], params['course_table'],
                                uids, cids)
    p = params
    row = lambda v: v.reshape(1, -1)
    w3p = jnp.pad(p['W3'], ((0, 7), (0, 0)))
    b3p = jnp.pad(p['b3'], (0, 7)).reshape(1, 8)
    wb = (p['W0'][:, :_EMB], p['W0'][:, _EMB:], p['W1'], p['W2'], w3p,
          row(p['b0']), row(p['b1']), row(p['b2']), b3p,
          row(p['gamma0']), row(p['gamma1']), row(p['gamma2']),
          row(p['beta0']), row(p['beta1']), row(p['beta2']))
    out = _mlp_call(u_emb, c_emb, wb)
    return out[:, 0]


# speedup vs baseline: 1.9968x; 1.9847x over previous
import functools

import numpy as np
import jax
import jax.numpy as jnp
from jax import lax
from jax.experimental import pallas as pl
from jax.experimental.pallas import tpu as pltpu
from jax.experimental.pallas import tpu_sc as plsc

_B = 16384
_EMB = 64
_NU = 1000000
_NCR = 100000
_HU = 503808
_HC = 53248

_NC, _NS = 2, 16
_NW = _NC * _NS
_BPW = _B // _NW
_CH = 128
_NCHUNK = _BPW // _CH

_PACK_R = 4096
_TILE = 2048


def _pack_body(t1_ref, t2_ref, out_ref):
    eye = (lax.broadcasted_iota(jnp.int32, (_EMB, _EMB), 0) ==
           lax.broadcasted_iota(jnp.int32, (_EMB, _EMB), 1)).astype(jnp.float32)

    def tr(x):
        return lax.dot_general(x, eye, (((0,), (0,)), ((), ())),
                               preferred_element_type=jnp.float32)

    out_ref[...] = jnp.concatenate([tr(t1_ref[...]), tr(t2_ref[...])], axis=1)


def _pack_call(tab_t, h):
    grid = h // _PACK_R
    hb = h // _PACK_R
    last = (tab_t.shape[1] + _PACK_R - 1) // _PACK_R - 1
    return pl.pallas_call(
        _pack_body,
        grid=(grid,),
        in_specs=[
            pl.BlockSpec((_EMB, _PACK_R), lambda i: (0, i)),
            pl.BlockSpec((_EMB, _PACK_R),
                         lambda i: (0, jnp.minimum(i + hb, last))),
        ],
        out_specs=pl.BlockSpec((_PACK_R, 2 * _EMB), lambda i: (i, 0)),
        out_shape=jax.ShapeDtypeStruct((h, 2 * _EMB), jnp.float32),
    )(tab_t, tab_t)


def _sc_gather_body(u_tab, c_tab, uids, cids, u_out, c_out,
                    idx_u, idx_c, rows, sem):
    wid = lax.axis_index("s") * _NC + lax.axis_index("c")
    base = wid * _BPW
    pltpu.sync_copy(uids.at[wid], idx_u)
    pltpu.sync_copy(cids.at[wid], idx_c)
    for tab, idx, out in ((u_tab, idx_u, u_out), (c_tab, idx_c, c_out)):
        copies = [pltpu.async_copy(
            tab.at[idx.at[j]], rows.at[pl.ds(j * _CH, _CH)], sem)
            for j in range(_NCHUNK)]
        for cp in copies:
            cp.wait()
        pltpu.sync_copy(rows, out.at[pl.ds(base, _BPW)])


@functools.cache
def _sc_gather():
    return pl.kernel(
        _sc_gather_body,
        out_type=(jax.ShapeDtypeStruct((_B, 2 * _EMB), jnp.float32),
                  jax.ShapeDtypeStruct((_B, 2 * _EMB), jnp.float32)),
        mesh=plsc.VectorSubcoreMesh(core_axis_name="c", subcore_axis_name="s"),
        compiler_params=pltpu.CompilerParams(use_tc_tiling_on_sc=True),
        scratch_types=[
            pltpu.VMEM((_NCHUNK, _CH), jnp.int32),
            pltpu.VMEM((_NCHUNK, _CH), jnp.int32),
            pltpu.VMEM((_BPW, 2 * _EMB), jnp.float32),
            pltpu.SemaphoreType.DMA,
        ],
    )


def _mlp_body(u_ref, c_ref, up_ref, cp_ref, w0u, w0c, w1, w2, w3,
              b0, b1, b2, b3, g0, g1, g2, be0, be1, be2, out_ref):
    s = np.float32(1.0 / np.sqrt(1.0 + 1e-5))

    def dot_t(x, w):
        return lax.dot_general(x, w, (((1,), (1,)), ((), ())),
                               preferred_element_type=jnp.float32)

    up = up_ref[...]
    cp = cp_ref[...]
    u2 = u_ref[...]
    c2 = c_ref[...]
    u = jnp.where(up > 0.5, u2[:, _EMB:], u2[:, :_EMB])
    c = jnp.where(cp > 0.5, c2[:, _EMB:], c2[:, :_EMB])
    h = dot_t(u, w0u[...]) + dot_t(c, w0c[...])
    h = jnp.maximum(h + b0[...], 0.0) * (g0[...] * s) + be0[...]
    h = dot_t(h, w1[...])
    h = jnp.maximum(h + b1[...], 0.0) * (g1[...] * s) + be1[...]
    h = dot_t(h, w2[...])
    h = jnp.maximum(h + b2[...], 0.0) * (g2[...] * s) + be2[...]
    h = dot_t(h, w3[...]) + b3[...]
    out_ref[...] = jax.nn.sigmoid(h)


def _mlp_call(u_emb, c_emb, upar, cpar, wb):
    full = lambda shape: pl.BlockSpec(shape, lambda i: (0, 0))
    return pl.pallas_call(
        _mlp_body,
        grid=(_B // _TILE,),
        in_specs=[
            pl.BlockSpec((_TILE, 2 * _EMB), lambda i: (i, 0)),
            pl.BlockSpec((_TILE, 2 * _EMB), lambda i: (i, 0)),
            pl.BlockSpec((_TILE, 1), lambda i: (i, 0)),
            pl.BlockSpec((_TILE, 1), lambda i: (i, 0)),
        ] + [full(w.shape) for w in wb],
        out_specs=pl.BlockSpec((_TILE, 8), lambda i: (i, 0)),
        out_shape=jax.ShapeDtypeStruct((_B, 8), jnp.float32),
    )(u_emb, c_emb, upar, cpar, *wb)


def kernel(user_ids, course_ids, params):
    user_ids = jnp.asarray(user_ids, jnp.int32)
    course_ids = jnp.asarray(course_ids, jnp.int32)
    u_pair = _pack_call(params['user_table'].T, _HU)
    c_pair = _pack_call(params['course_table'].T, _HC)
    uhalf = (user_ids >= _HU).astype(jnp.int32)
    chalf = (course_ids >= _HC).astype(jnp.int32)
    uids = (user_ids - uhalf * _HU).reshape(_NW, _NCHUNK, _CH)
    cids = (course_ids - chalf * _HC).reshape(_NW, _NCHUNK, _CH)
    u_emb, c_emb = _sc_gather()(u_pair, c_pair, uids, cids)
    upar = uhalf.astype(jnp.float32).reshape(_B, 1)
    cpar = chalf.astype(jnp.float32).reshape(_B, 1)
    p = params
    row = lambda v: v.reshape(1, -1)
    w3p = jnp.pad(p['W3'], ((0, 7), (0, 0)))
    b3p = jnp.pad(p['b3'], (0, 7)).reshape(1, 8)
    wb = (p['W0'][:, :_EMB], p['W0'][:, _EMB:], p['W1'], p['W2'], w3p,
          row(p['b0']), row(p['b1']), row(p['b2']), b3p,
          row(p['gamma0']), row(p['gamma1']), row(p['gamma2']),
          row(p['beta0']), row(p['beta1']), row(p['beta2']))
    out = _mlp_call(u_emb, c_emb, upar, cpar, wb)
    return out[:, 0]


# speedup vs baseline: 2.1298x; 1.0666x over previous
import functools

import numpy as np
import jax
import jax.numpy as jnp
from jax import lax
from jax.experimental import pallas as pl
from jax.experimental.pallas import tpu as pltpu
from jax.experimental.pallas import tpu_sc as plsc

_B = 16384
_EMB = 64
_PACK_R = 4096
_HU = 253952
_HC = 28672

_NC, _NS = 2, 16
_NW = _NC * _NS
_BPW = _B // _NW
_CH = 128
_NCHUNK = _BPW // _CH

_TILE = 2048


def _pack_body(t0_ref, t1_ref, t2_ref, t3_ref, out_ref):
    eye = (lax.broadcasted_iota(jnp.int32, (_EMB, _EMB), 0) ==
           lax.broadcasted_iota(jnp.int32, (_EMB, _EMB), 1)).astype(jnp.float32)

    def tr_bits(ref):
        t = lax.dot_general(ref[...], eye, (((0,), (0,)), ((), ())),
                            preferred_element_type=jnp.float32)
        bits = lax.bitcast_convert_type(t, jnp.int32)
        return ((bits + 0x8000) >> 16) & 0xFFFF

    lo = jnp.concatenate([tr_bits(t0_ref), tr_bits(t2_ref)], axis=1)
    hi = jnp.concatenate([tr_bits(t1_ref), tr_bits(t3_ref)], axis=1)
    out_ref[...] = (hi << 16) | lo


def _pack_call(tab_t, h):
    grid = h // _PACK_R
    hb = h // _PACK_R
    last = (tab_t.shape[1] + _PACK_R - 1) // _PACK_R - 1
    spec = lambda k: pl.BlockSpec(
        (_EMB, _PACK_R), lambda i: (0, jnp.minimum(i + k * hb, last)))
    return pl.pallas_call(
        _pack_body,
        grid=(grid,),
        in_specs=[pl.BlockSpec((_EMB, _PACK_R), lambda i: (0, i)),
                  spec(1), spec(2), spec(3)],
        out_specs=pl.BlockSpec((_PACK_R, 2 * _EMB), lambda i: (i, 0)),
        out_shape=jax.ShapeDtypeStruct((h, 2 * _EMB), jnp.int32),
    )(tab_t, tab_t, tab_t, tab_t)


def _sc_gather_body(u_tab, c_tab, uids, cids, u_out, c_out,
                    idx_u, idx_c, rows, sem):
    wid = lax.axis_index("s") * _NC + lax.axis_index("c")
    base = wid * _BPW
    pltpu.sync_copy(uids.at[wid], idx_u)
    pltpu.sync_copy(cids.at[wid], idx_c)
    for tab, idx, out in ((u_tab, idx_u, u_out), (c_tab, idx_c, c_out)):
        copies = [pltpu.async_copy(
            tab.at[idx.at[j]], rows.at[pl.ds(j * _CH, _CH)], sem)
            for j in range(_NCHUNK)]
        for cp in copies:
            cp.wait()
        pltpu.sync_copy(rows, out.at[pl.ds(base, _BPW)])


@functools.cache
def _sc_gather():
    return pl.kernel(
        _sc_gather_body,
        out_type=(jax.ShapeDtypeStruct((_B, 2 * _EMB), jnp.int32),
                  jax.ShapeDtypeStruct((_B, 2 * _EMB), jnp.int32)),
        mesh=plsc.VectorSubcoreMesh(core_axis_name="c", subcore_axis_name="s"),
        compiler_params=pltpu.CompilerParams(use_tc_tiling_on_sc=True),
        scratch_types=[
            pltpu.VMEM((_NCHUNK, _CH), jnp.int32),
            pltpu.VMEM((_NCHUNK, _CH), jnp.int32),
            pltpu.VMEM((_BPW, 2 * _EMB), jnp.int32),
            pltpu.SemaphoreType.DMA,
        ],
    )


def _mlp_body(u_ref, c_ref, ub0_ref, ub1_ref, cb0_ref, cb1_ref,
              w0u, w0c, w1, w2, w3,
              b0, b1, b2, b3, g0, g1, g2, be0, be1, be2, out_ref):
    s = np.float32(1.0 / np.sqrt(1.0 + 1e-5))

    def dot_t(x, w):
        return lax.dot_general(x, w, (((1,), (1,)), ((), ())),
                               preferred_element_type=jnp.float32)

    def select(q_ref, sb0, sb1):
        q = q_ref[...]
        w = jnp.where(sb1 > 0, q[:, _EMB:], q[:, :_EMB])
        bits = jnp.where(sb0 > 0, w & jnp.int32(-65536), w << 16)
        return lax.bitcast_convert_type(bits, jnp.float32)

    u = select(u_ref, ub0_ref[...], ub1_ref[...])
    c = select(c_ref, cb0_ref[...], cb1_ref[...])
    h = dot_t(u, w0u[...]) + dot_t(c, w0c[...])
    h = jnp.maximum(h + b0[...], 0.0) * (g0[...] * s) + be0[...]
    h = dot_t(h, w1[...])
    h = jnp.maximum(h + b1[...], 0.0) * (g1[...] * s) + be1[...]
    h = dot_t(h, w2[...])
    h = jnp.maximum(h + b2[...], 0.0) * (g2[...] * s) + be2[...]
    h = dot_t(h, w3[...]) + b3[...]
    out_ref[...] = jax.nn.sigmoid(h)


def _mlp_call(u_emb, c_emb, sel, wb):
    full = lambda shape: pl.BlockSpec(shape, lambda i: (0, 0))
    return pl.pallas_call(
        _mlp_body,
        grid=(_B // _TILE,),
        in_specs=[
            pl.BlockSpec((_TILE, 2 * _EMB), lambda i: (i, 0)),
            pl.BlockSpec((_TILE, 2 * _EMB), lambda i: (i, 0)),
        ] + [pl.BlockSpec((_TILE, 1), lambda i: (i, 0))] * 4
          + [full(w.shape) for w in wb],
        out_specs=pl.BlockSpec((_TILE, 8), lambda i: (i, 0)),
        out_shape=jax.ShapeDtypeStruct((_B, 8), jnp.float32),
    )(u_emb, c_emb, *sel, *wb)


def kernel(user_ids, course_ids, params):
    user_ids = jnp.asarray(user_ids, jnp.int32)
    course_ids = jnp.asarray(course_ids, jnp.int32)
    u_quad = _pack_call(params['user_table'].T, _HU)
    c_quad = _pack_call(params['course_table'].T, _HC)
    uj = user_ids // _HU
    cj = course_ids // _HC
    uids = (user_ids - uj * _HU).reshape(_NW, _NCHUNK, _CH)
    cids = (course_ids - cj * _HC).reshape(_NW, _NCHUNK, _CH)
    u_emb, c_emb = _sc_gather()(u_quad, c_quad, uids, cids)
    f = lambda v: v.astype(jnp.int32).reshape(_B, 1)
    sel = (f(uj & 1), f(uj >> 1), f(cj & 1), f(cj >> 1))
    p = params
    row = lambda v: v.reshape(1, -1)
    w3p = jnp.pad(p['W3'], ((0, 7), (0, 0)))
    b3p = jnp.pad(p['b3'], (0, 7)).reshape(1, 8)
    wb = (p['W0'][:, :_EMB], p['W0'][:, _EMB:], p['W1'], p['W2'], w3p,
          row(p['b0']), row(p['b1']), row(p['b2']), b3p,
          row(p['gamma0']), row(p['gamma1']), row(p['gamma2']),
          row(p['beta0']), row(p['beta1']), row(p['beta2']))
    out = _mlp_call(u_emb, c_emb, sel, wb)
    return out[:, 0]


# speedup vs baseline: 3.1192x; 1.4646x over previous
import functools

import numpy as np
import jax
import jax.numpy as jnp
from jax import lax
from jax.experimental import pallas as pl
from jax.experimental.pallas import tpu as pltpu
from jax.experimental.pallas import tpu_sc as plsc

_B = 16384
_EMB = 64
_PACK_R = 4096
_HU = 253952
_HC = 28672

_NC, _NS = 2, 16
_NW = _NC * _NS
_BPW = _B // _NW
_CH = 128
_NCHUNK = _BPW // _CH

_TILE = 2048


def _pack_body(t0_ref, t1_ref, t2_ref, t3_ref, out_ref):
    def bits16(ref):
        b = lax.bitcast_convert_type(ref[...], jnp.int32)
        return ((b + 0x8000) >> 16) & 0xFFFF

    w01 = (bits16(t1_ref) << 16) | bits16(t0_ref)
    w23 = (bits16(t3_ref) << 16) | bits16(t2_ref)
    w = jnp.concatenate([w01, w23], axis=0)
    out_ref[...] = w.T


def _pack_call(tab_t, h):
    grid = h // _PACK_R
    hb = h // _PACK_R
    last = (tab_t.shape[1] + _PACK_R - 1) // _PACK_R - 1
    spec = lambda k: pl.BlockSpec(
        (_EMB, _PACK_R), lambda i: (0, jnp.minimum(i + k * hb, last)))
    return pl.pallas_call(
        _pack_body,
        grid=(grid,),
        in_specs=[pl.BlockSpec((_EMB, _PACK_R), lambda i: (0, i)),
                  spec(1), spec(2), spec(3)],
        out_specs=pl.BlockSpec((_PACK_R, 2 * _EMB), lambda i: (i, 0)),
        out_shape=jax.ShapeDtypeStruct((h, 2 * _EMB), jnp.int32),
    )(tab_t, tab_t, tab_t, tab_t)


def _sc_gather_body(u_tab, c_tab, uids, cids, u_out, c_out,
                    idx_u, idx_c, rows, sem):
    wid = lax.axis_index("s") * _NC + lax.axis_index("c")
    base = wid * _BPW
    pltpu.sync_copy(uids.at[wid], idx_u)
    pltpu.sync_copy(cids.at[wid], idx_c)
    for tab, idx, out in ((u_tab, idx_u, u_out), (c_tab, idx_c, c_out)):
        copies = [pltpu.async_copy(
            tab.at[idx.at[j]], rows.at[pl.ds(j * _CH, _CH)], sem)
            for j in range(_NCHUNK)]
        for cp in copies:
            cp.wait()
        pltpu.sync_copy(rows, out.at[pl.ds(base, _BPW)])


@functools.cache
def _sc_gather():
    return pl.kernel(
        _sc_gather_body,
        out_type=(jax.ShapeDtypeStruct((_B, 2 * _EMB), jnp.int32),
                  jax.ShapeDtypeStruct((_B, 2 * _EMB), jnp.int32)),
        mesh=plsc.VectorSubcoreMesh(core_axis_name="c", subcore_axis_name="s"),
        compiler_params=pltpu.CompilerParams(use_tc_tiling_on_sc=True),
        scratch_types=[
            pltpu.VMEM((_NCHUNK, _CH), jnp.int32),
            pltpu.VMEM((_NCHUNK, _CH), jnp.int32),
            pltpu.VMEM((_BPW, 2 * _EMB), jnp.int32),
            pltpu.SemaphoreType.DMA,
        ],
    )


def _mlp_body(u_ref, c_ref, ub0_ref, ub1_ref, cb0_ref, cb1_ref,
              w0u, w0c, w1, w2, w3,
              b0, b1, b2, b3, g0, g1, g2, be0, be1, be2, out_ref):
    s = np.float32(1.0 / np.sqrt(1.0 + 1e-5))

    def dot_t(x, w):
        return lax.dot_general(x, w, (((1,), (1,)), ((), ())),
                               preferred_element_type=jnp.float32)

    def select(q_ref, sb0, sb1):
        q = q_ref[...]
        w = jnp.where(sb1 > 0, q[:, _EMB:], q[:, :_EMB])
        bits = jnp.where(sb0 > 0, w & jnp.int32(-65536), w << 16)
        return lax.bitcast_convert_type(bits, jnp.float32)

    u = select(u_ref, ub0_ref[...], ub1_ref[...])
    c = select(c_ref, cb0_ref[...], cb1_ref[...])
    h = dot_t(u, w0u[...]) + dot_t(c, w0c[...])
    h = jnp.maximum(h + b0[...], 0.0) * (g0[...] * s) + be0[...]
    h = dot_t(h, w1[...])
    h = jnp.maximum(h + b1[...], 0.0) * (g1[...] * s) + be1[...]
    h = dot_t(h, w2[...])
    h = jnp.maximum(h + b2[...], 0.0) * (g2[...] * s) + be2[...]
    h = dot_t(h, w3[...]) + b3[...]
    out_ref[...] = jax.nn.sigmoid(h)


def _mlp_call(u_emb, c_emb, sel, wb):
    full = lambda shape: pl.BlockSpec(shape, lambda i: (0, 0))
    return pl.pallas_call(
        _mlp_body,
        grid=(_B // _TILE,),
        in_specs=[
            pl.BlockSpec((_TILE, 2 * _EMB), lambda i: (i, 0)),
            pl.BlockSpec((_TILE, 2 * _EMB), lambda i: (i, 0)),
        ] + [pl.BlockSpec((_TILE, 1), lambda i: (i, 0))] * 4
          + [full(w.shape) for w in wb],
        out_specs=pl.BlockSpec((_TILE, 8), lambda i: (i, 0)),
        out_shape=jax.ShapeDtypeStruct((_B, 8), jnp.float32),
    )(u_emb, c_emb, *sel, *wb)


def kernel(user_ids, course_ids, params):
    user_ids = jnp.asarray(user_ids, jnp.int32)
    course_ids = jnp.asarray(course_ids, jnp.int32)
    u_quad = _pack_call(params['user_table'].T, _HU)
    c_quad = _pack_call(params['course_table'].T, _HC)
    uj = user_ids // _HU
    cj = course_ids // _HC
    uids = (user_ids - uj * _HU).reshape(_NW, _NCHUNK, _CH)
    cids = (course_ids - cj * _HC).reshape(_NW, _NCHUNK, _CH)
    u_emb, c_emb = _sc_gather()(u_quad, c_quad, uids, cids)
    f = lambda v: v.astype(jnp.int32).reshape(_B, 1)
    sel = (f(uj & 1), f(uj >> 1), f(cj & 1), f(cj >> 1))
    p = params
    row = lambda v: v.reshape(1, -1)
    w3p = jnp.pad(p['W3'], ((0, 7), (0, 0)))
    b3p = jnp.pad(p['b3'], (0, 7)).reshape(1, 8)
    wb = (p['W0'][:, :_EMB], p['W0'][:, _EMB:], p['W1'], p['W2'], w3p,
          row(p['b0']), row(p['b1']), row(p['b2']), b3p,
          row(p['gamma0']), row(p['gamma1']), row(p['gamma2']),
          row(p['beta0']), row(p['beta1']), row(p['beta2']))
    out = _mlp_call(u_emb, c_emb, sel, wb)
    return out[:, 0]


# speedup vs baseline: 3.1336x; 1.0046x over previous
import functools

import numpy as np
import jax
import jax.numpy as jnp
from jax import lax
from jax.experimental import pallas as pl
from jax.experimental.pallas import tpu as pltpu
from jax.experimental.pallas import tpu_sc as plsc

_B = 16384
_EMB = 64
_PACK_R = 4096
_HU = 253952
_HC = 28672

_NC, _NS = 2, 16
_NW = _NC * _NS
_BPW = _B // _NW
_CH = 128
_NCHUNK = _BPW // _CH

_TILE = 4096


def _pack_body(t0_ref, t1_ref, t2_ref, t3_ref, out_ref):
    def bits16(ref):
        b = lax.bitcast_convert_type(ref[...], jnp.int32)
        return ((b + 0x8000) >> 16) & 0xFFFF

    w01 = (bits16(t1_ref) << 16) | bits16(t0_ref)
    w23 = (bits16(t3_ref) << 16) | bits16(t2_ref)
    w = jnp.concatenate([w01, w23], axis=0)
    out_ref[...] = w.T


def _pack_call(tab_t, h):
    grid = h // _PACK_R
    hb = h // _PACK_R
    last = (tab_t.shape[1] + _PACK_R - 1) // _PACK_R - 1
    spec = lambda k: pl.BlockSpec(
        (_EMB, _PACK_R), lambda i: (0, jnp.minimum(i + k * hb, last)))
    return pl.pallas_call(
        _pack_body,
        grid=(grid,),
        in_specs=[pl.BlockSpec((_EMB, _PACK_R), lambda i: (0, i)),
                  spec(1), spec(2), spec(3)],
        out_specs=pl.BlockSpec((_PACK_R, 2 * _EMB), lambda i: (i, 0)),
        out_shape=jax.ShapeDtypeStruct((h, 2 * _EMB), jnp.int32),
    )(tab_t, tab_t, tab_t, tab_t)


def _sc_gather_body(u_tab, c_tab, uids, cids, u_out, c_out,
                    idx_u, idx_c, rows, sem):
    wid = lax.axis_index("s") * _NC + lax.axis_index("c")
    base = wid * _BPW
    pltpu.sync_copy(uids.at[wid], idx_u)
    pltpu.sync_copy(cids.at[wid], idx_c)
    for tab, idx, out in ((u_tab, idx_u, u_out), (c_tab, idx_c, c_out)):
        copies = [pltpu.async_copy(
            tab.at[idx.at[j]], rows.at[pl.ds(j * _CH, _CH)], sem)
            for j in range(_NCHUNK)]
        for cp in copies:
            cp.wait()
        pltpu.sync_copy(rows, out.at[pl.ds(base, _BPW)])


@functools.cache
def _sc_gather():
    return pl.kernel(
        _sc_gather_body,
        out_type=(jax.ShapeDtypeStruct((_B, 2 * _EMB), jnp.int32),
                  jax.ShapeDtypeStruct((_B, 2 * _EMB), jnp.int32)),
        mesh=plsc.VectorSubcoreMesh(core_axis_name="c", subcore_axis_name="s"),
        compiler_params=pltpu.CompilerParams(use_tc_tiling_on_sc=True),
        scratch_types=[
            pltpu.VMEM((_NCHUNK, _CH), jnp.int32),
            pltpu.VMEM((_NCHUNK, _CH), jnp.int32),
            pltpu.VMEM((_BPW, 2 * _EMB), jnp.int32),
            pltpu.SemaphoreType.DMA,
        ],
    )


def _mlp_body(u_ref, c_ref, ub0_ref, ub1_ref, cb0_ref, cb1_ref,
              w0u, w0c, w1, w2, w3,
              b0, b1, b2, b3, g0, g1, g2, be0, be1, be2, out_ref):
    s = np.float32(1.0 / np.sqrt(1.0 + 1e-5))

    def dot_t(x, w):
        return lax.dot_general(x, w, (((1,), (0,)), ((), ())),
                               preferred_element_type=jnp.float32)

    def select(q_ref, sb0, sb1):
        q = q_ref[...]
        w = jnp.where(sb1 > 0, q[:, _EMB:], q[:, :_EMB])
        bits = jnp.where(sb0 > 0, w & jnp.int32(-65536), w << 16)
        return lax.bitcast_convert_type(bits, jnp.float32)

    u = select(u_ref, ub0_ref[...], ub1_ref[...])
    c = select(c_ref, cb0_ref[...], cb1_ref[...])
    h = dot_t(u, w0u[...]) + dot_t(c, w0c[...])
    h = jnp.maximum(h + b0[...], 0.0) * (g0[...] * s) + be0[...]
    h = dot_t(h, w1[...])
    h = jnp.maximum(h + b1[...], 0.0) * (g1[...] * s) + be1[...]
    h = dot_t(h, w2[...])
    h = jnp.maximum(h + b2[...], 0.0) * (g2[...] * s) + be2[...]
    h = dot_t(h, w3[...]) + b3[...]
    out_ref[...] = jax.nn.sigmoid(h)


def _mlp_call(u_emb, c_emb, sel, wb):
    full = lambda shape: pl.BlockSpec(shape, lambda i: (0, 0))
    return pl.pallas_call(
        _mlp_body,
        grid=(_B // _TILE,),
        in_specs=[
            pl.BlockSpec((_TILE, 2 * _EMB), lambda i: (i, 0)),
            pl.BlockSpec((_TILE, 2 * _EMB), lambda i: (i, 0)),
        ] + [pl.BlockSpec((_TILE, 1), lambda i: (i, 0))] * 4
          + [full(w.shape) for w in wb],
        out_specs=pl.BlockSpec((_TILE, 8), lambda i: (i, 0)),
        out_shape=jax.ShapeDtypeStruct((_B, 8), jnp.float32),
    )(u_emb, c_emb, *sel, *wb)


def kernel(user_ids, course_ids, params):
    user_ids = jnp.asarray(user_ids, jnp.int32)
    course_ids = jnp.asarray(course_ids, jnp.int32)
    u_quad = _pack_call(params['user_table'].T, _HU)
    c_quad = _pack_call(params['course_table'].T, _HC)
    uj = user_ids // _HU
    cj = course_ids // _HC
    uids = (user_ids - uj * _HU).reshape(_NW, _NCHUNK, _CH)
    cids = (course_ids - cj * _HC).reshape(_NW, _NCHUNK, _CH)
    u_emb, c_emb = _sc_gather()(u_quad, c_quad, uids, cids)
    f = lambda v: v.astype(jnp.int32).reshape(_B, 1)
    sel = (f(uj & 1), f(uj >> 1), f(cj & 1), f(cj >> 1))
    p = params
    row = lambda v: v.reshape(1, -1)
    w3p = jnp.pad(p['W3'], ((0, 7), (0, 0)))
    b3p = jnp.pad(p['b3'], (0, 7)).reshape(1, 8)
    wb = (p['W0'][:, :_EMB].T, p['W0'][:, _EMB:].T, p['W1'].T, p['W2'].T,
          w3p.T,
          row(p['b0']), row(p['b1']), row(p['b2']), b3p,
          row(p['gamma0']), row(p['gamma1']), row(p['gamma2']),
          row(p['beta0']), row(p['beta1']), row(p['beta2']))
    out = _mlp_call(u_emb, c_emb, sel, wb)
    return out[:, 0]


# speedup vs baseline: 3.3587x; 1.0719x over previous
import functools

import numpy as np
import jax
import jax.numpy as jnp
from jax import lax
from jax.experimental import pallas as pl
from jax.experimental.pallas import tpu as pltpu
from jax.experimental.pallas import tpu_sc as plsc

_B = 16384
_EMB = 64
_PACK_R = 4096
_HU = 253952
_HC = 28672

_NC, _NS = 2, 16
_NW = _NC * _NS
_BPW = _B // _NW
_CH = 128
_NCHUNK = _BPW // _CH

_TILE = 4096


def _pack_body(t0_ref, t1_ref, t2_ref, t3_ref, out_ref):
    def bits16(ref):
        b = lax.bitcast_convert_type(ref[...], jnp.int32)
        return ((b + 0x8000) >> 16) & 0xFFFF

    w01 = (bits16(t1_ref) << 16) | bits16(t0_ref)
    w23 = (bits16(t3_ref) << 16) | bits16(t2_ref)
    w = jnp.concatenate([w01, w23], axis=0)
    out_ref[...] = w.T


def _pack_call(tab_t, h):
    grid = h // _PACK_R
    hb = h // _PACK_R
    last = (tab_t.shape[1] + _PACK_R - 1) // _PACK_R - 1
    spec = lambda k: pl.BlockSpec(
        (_EMB, _PACK_R), lambda i: (0, jnp.minimum(i + k * hb, last)))
    return pl.pallas_call(
        _pack_body,
        grid=(grid,),
        in_specs=[pl.BlockSpec((_EMB, _PACK_R), lambda i: (0, i)),
                  spec(1), spec(2), spec(3)],
        out_specs=pl.BlockSpec((_PACK_R, 2 * _EMB), lambda i: (i, 0)),
        out_shape=jax.ShapeDtypeStruct((h, 2 * _EMB), jnp.int32),
    )(tab_t, tab_t, tab_t, tab_t)


def _sc_gather_body(u_tab, c_tab, uids, cids, u_out, c_out,
                    idx_u, idx_c, rows, sem):
    wid = lax.axis_index("s") * _NC + lax.axis_index("c")
    base = wid * _BPW
    pltpu.sync_copy(uids.at[wid], idx_u)
    pltpu.sync_copy(cids.at[wid], idx_c)
    for tab, idx, out in ((u_tab, idx_u, u_out), (c_tab, idx_c, c_out)):
        copies = [pltpu.async_copy(
            tab.at[idx.at[j]], rows.at[pl.ds(j * _CH, _CH)], sem)
            for j in range(_NCHUNK)]
        for cp in copies:
            cp.wait()
        pltpu.sync_copy(rows, out.at[pl.ds(base, _BPW)])


@functools.cache
def _sc_gather():
    return pl.kernel(
        _sc_gather_body,
        out_type=(jax.ShapeDtypeStruct((_B, 2 * _EMB), jnp.int32),
                  jax.ShapeDtypeStruct((_B, 2 * _EMB), jnp.int32)),
        mesh=plsc.VectorSubcoreMesh(core_axis_name="c", subcore_axis_name="s"),
        compiler_params=pltpu.CompilerParams(use_tc_tiling_on_sc=True),
        scratch_types=[
            pltpu.VMEM((_NCHUNK, _CH), jnp.int32),
            pltpu.VMEM((_NCHUNK, _CH), jnp.int32),
            pltpu.VMEM((_BPW, 2 * _EMB), jnp.int32),
            pltpu.SemaphoreType.DMA,
        ],
    )


def _mlp_body(u_ref, c_ref, code_ref,
              w0u, w0c, w1, w2, w3,
              b0, b1, b2, b3, g0, g1, g2, be0, be1, be2, out_ref):
    s = np.float32(1.0 / np.sqrt(1.0 + 1e-5))

    def dot_t(x, w):
        return lax.dot_general(x, w, (((1,), (0,)), ((), ())),
                               preferred_element_type=jnp.float32)

    code = code_ref[...][:, :1]

    def select(q_ref, shift):
        q = q_ref[...]
        sb0 = (code >> shift) & 1
        sb1 = (code >> (shift + 1)) & 1
        w = jnp.where(sb1 > 0, q[:, _EMB:], q[:, :_EMB])
        bits = jnp.where(sb0 > 0, w & jnp.int32(-65536), w << 16)
        return lax.bitcast_convert_type(bits, jnp.float32)

    u = select(u_ref, 0)
    c = select(c_ref, 2)
    h = dot_t(u, w0u[...]) + dot_t(c, w0c[...])
    h = jnp.maximum(h + b0[...], 0.0) * (g0[...] * s) + be0[...]
    h = dot_t(h, w1[...])
    h = jnp.maximum(h + b1[...], 0.0) * (g1[...] * s) + be1[...]
    h = dot_t(h, w2[...])
    h = jnp.maximum(h + b2[...], 0.0) * (g2[...] * s) + be2[...]
    h = dot_t(h, w3[...]) + b3[...]
    out_ref[...] = lax.transpose(jax.nn.sigmoid(h), (1, 0))


def _mlp_call(u_emb, c_emb, code, wb):
    full = lambda shape: pl.BlockSpec(shape, lambda i: (0, 0))
    return pl.pallas_call(
        _mlp_body,
        grid=(_B // _TILE,),
        in_specs=[
            pl.BlockSpec((_TILE, 2 * _EMB), lambda i: (i, 0)),
            pl.BlockSpec((_TILE, 2 * _EMB), lambda i: (i, 0)),
            pl.BlockSpec((_TILE, 8), lambda i: (i, 0)),
        ] + [full(w.shape) for w in wb],
        out_specs=pl.BlockSpec((8, _TILE), lambda i: (0, i)),
        out_shape=jax.ShapeDtypeStruct((8, _B), jnp.float32),
    )(u_emb, c_emb, code, *wb)


def kernel(user_ids, course_ids, params):
    user_ids = jnp.asarray(user_ids, jnp.int32)
    course_ids = jnp.asarray(course_ids, jnp.int32)
    u_quad = _pack_call(params['user_table'].T, _HU)
    c_quad = _pack_call(params['course_table'].T, _HC)
    uj = user_ids // _HU
    cj = course_ids // _HC
    uids = (user_ids - uj * _HU).reshape(_NW, _NCHUNK, _CH)
    cids = (course_ids - cj * _HC).reshape(_NW, _NCHUNK, _CH)
    u_emb, c_emb = _sc_gather()(u_quad, c_quad, uids, cids)
    code = jnp.broadcast_to((uj | (cj << 2)).reshape(_B, 1), (_B, 8))
    p = params
    row = lambda v: v.reshape(1, -1)
    w3p = jnp.pad(p['W3'], ((0, 7), (0, 0)))
    b3p = jnp.pad(p['b3'], (0, 7)).reshape(1, 8)
    wb = (p['W0'][:, :_EMB].T, p['W0'][:, _EMB:].T, p['W1'].T, p['W2'].T,
          w3p.T,
          row(p['b0']), row(p['b1']), row(p['b2']), b3p,
          row(p['gamma0']), row(p['gamma1']), row(p['gamma2']),
          row(p['beta0']), row(p['beta1']), row(p['beta2']))
    out = _mlp_call(u_emb, c_emb, code, wb)
    return out[0]


# speedup vs baseline: 3.3807x; 1.0065x over previous
import functools

import numpy as np
import jax
import jax.numpy as jnp
from jax import lax
from jax.experimental import pallas as pl
from jax.experimental.pallas import tpu as pltpu
from jax.experimental.pallas import tpu_sc as plsc

_B = 16384
_EMB = 64
_PACK_R = 4096
_HU = 253952
_HC = 28672

_NC, _NS = 2, 16
_NW = _NC * _NS
_BPW = _B // _NW
_CH = 128
_NCHUNK = _BPW // _CH

_TILE = 4096


def _pack_body(t0_ref, t1_ref, t2_ref, t3_ref, out_ref):
    def bits16(ref):
        b = lax.bitcast_convert_type(ref[...], jnp.int32)
        return ((b + 0x8000) >> 16) & 0xFFFF

    w01 = (bits16(t1_ref) << 16) | bits16(t0_ref)
    w23 = (bits16(t3_ref) << 16) | bits16(t2_ref)
    w = jnp.concatenate([w01, w23], axis=0)
    out_ref[...] = w.T


def _pack_call(tab_t, h):
    grid = h // _PACK_R
    hb = h // _PACK_R
    last = (tab_t.shape[1] + _PACK_R - 1) // _PACK_R - 1
    spec = lambda k: pl.BlockSpec(
        (_EMB, _PACK_R), lambda i: (0, jnp.minimum(i + k * hb, last)))
    return pl.pallas_call(
        _pack_body,
        grid=(grid,),
        in_specs=[pl.BlockSpec((_EMB, _PACK_R), lambda i: (0, i)),
                  spec(1), spec(2), spec(3)],
        out_specs=pl.BlockSpec((_PACK_R, 2 * _EMB), lambda i: (i, 0)),
        out_shape=jax.ShapeDtypeStruct((h, 2 * _EMB), jnp.int32),
    )(tab_t, tab_t, tab_t, tab_t)


def _sc_gather_body(tab, ids, out, idx, rows, sem):
    wid = lax.axis_index("s") * _NC + lax.axis_index("c")
    base = wid * _BPW
    pltpu.sync_copy(ids.at[wid], idx)
    copies = [pltpu.async_copy(
        tab.at[idx.at[j]], rows.at[pl.ds(j * _CH, _CH)], sem)
        for j in range(_NCHUNK)]
    for cp in copies:
        cp.wait()
    pltpu.sync_copy(rows, out.at[pl.ds(base, _BPW)])


@functools.cache
def _sc_gather():
    return pl.kernel(
        _sc_gather_body,
        out_type=jax.ShapeDtypeStruct((_B, 2 * _EMB), jnp.int32),
        mesh=plsc.VectorSubcoreMesh(core_axis_name="c", subcore_axis_name="s"),
        compiler_params=pltpu.CompilerParams(use_tc_tiling_on_sc=True),
        scratch_types=[
            pltpu.VMEM((_NCHUNK, _CH), jnp.int32),
            pltpu.VMEM((_BPW, 2 * _EMB), jnp.int32),
            pltpu.SemaphoreType.DMA,
        ],
    )


def _mlp_body(u_ref, c_ref, code_ref,
              w0u, w0c, w1, w2, w3,
              b0, b1, b2, b3, g0, g1, g2, be0, be1, be2, out_ref):
    s = np.float32(1.0 / np.sqrt(1.0 + 1e-5))

    def dot_t(x, w):
        return lax.dot_general(x, w, (((1,), (0,)), ((), ())),
                               preferred_element_type=jnp.float32)

    code = code_ref[...][:, :1]

    def select(q_ref, shift):
        q = q_ref[...]
        sb0 = (code >> shift) & 1
        sb1 = (code >> (shift + 1)) & 1
        w = jnp.where(sb1 > 0, q[:, _EMB:], q[:, :_EMB])
        bits = jnp.where(sb0 > 0, w & jnp.int32(-65536), w << 16)
        return lax.bitcast_convert_type(bits, jnp.float32)

    u = select(u_ref, 0)
    c = select(c_ref, 2)
    h = dot_t(u, w0u[...]) + dot_t(c, w0c[...])
    h = jnp.maximum(h + b0[...], 0.0) * (g0[...] * s) + be0[...]
    h = dot_t(h, w1[...])
    h = jnp.maximum(h + b1[...], 0.0) * (g1[...] * s) + be1[...]
    h = dot_t(h, w2[...])
    h = jnp.maximum(h + b2[...], 0.0) * (g2[...] * s) + be2[...]
    h = dot_t(h, w3[...]) + b3[...]
    out_ref[...] = lax.transpose(jax.nn.sigmoid(h), (1, 0))


def _mlp_call(u_emb, c_emb, code, wb):
    full = lambda shape: pl.BlockSpec(shape, lambda i: (0, 0))
    return pl.pallas_call(
        _mlp_body,
        grid=(_B // _TILE,),
        in_specs=[
            pl.BlockSpec((_TILE, 2 * _EMB), lambda i: (i, 0)),
            pl.BlockSpec((_TILE, 2 * _EMB), lambda i: (i, 0)),
            pl.BlockSpec((_TILE, 8), lambda i: (i, 0)),
        ] + [full(w.shape) for w in wb],
        out_specs=pl.BlockSpec((8, _TILE), lambda i: (0, i)),
        out_shape=jax.ShapeDtypeStruct((8, _B), jnp.float32),
    )(u_emb, c_emb, code, *wb)


def kernel(user_ids, course_ids, params):
    user_ids = jnp.asarray(user_ids, jnp.int32)
    course_ids = jnp.asarray(course_ids, jnp.int32)
    u_quad = _pack_call(params['user_table'].T, _HU)
    c_quad = _pack_call(params['course_table'].T, _HC)
    uj = user_ids // _HU
    cj = course_ids // _HC
    uids = (user_ids - uj * _HU).reshape(_NW, _NCHUNK, _CH)
    cids = (course_ids - cj * _HC).reshape(_NW, _NCHUNK, _CH)
    u_emb = _sc_gather()(u_quad, uids)
    c_emb = _sc_gather()(c_quad, cids)
    code = jnp.broadcast_to((uj | (cj << 2)).reshape(_B, 1), (_B, 8))
    p = params
    row = lambda v: v.reshape(1, -1)
    w3p = jnp.pad(p['W3'], ((0, 7), (0, 0)))
    b3p = jnp.pad(p['b3'], (0, 7)).reshape(1, 8)
    wb = (p['W0'][:, :_EMB].T, p['W0'][:, _EMB:].T, p['W1'].T, p['W2'].T,
          w3p.T,
          row(p['b0']), row(p['b1']), row(p['b2']), b3p,
          row(p['gamma0']), row(p['gamma1']), row(p['gamma2']),
          row(p['beta0']), row(p['beta1']), row(p['beta2']))
    out = _mlp_call(u_emb, c_emb, code, wb)
    return out[0]
